# baseline (speedup 1.0000x reference)
import functools

import jax
import jax.numpy as jnp
from jax.experimental import pallas as pl
from jax.experimental.pallas import tpu as pltpu

N = 10000
E = 160000
B = 1024
D_IN = 128
HID = 128
OUT = 64


def _combine_body(osr_ref, feat_ref, w1_ref, b1_ref, w2_ref, b2_ref,
                  alpha_ref, out_ref, ostruct_ref):
    x = osr_ref[:]
    h = jnp.maximum(jnp.dot(x, w1_ref[:]) + b1_ref[:], 0.0)
    y = jnp.dot(h, w2_ref[:]) + b2_ref[:]
    os_ = jax.nn.sigmoid(y)
    al = alpha_ref[:]
    m = jnp.max(al)
    ea = jnp.exp(al - m)
    a = ea / jnp.sum(ea)
    ostruct_ref[:] = os_
    out_ref[:] = a[0, 0] * os_ + a[0, 1] * feat_ref[:] + 1e-15


def _combine(out_struct_raw, out_feat, gp_W1, gp_b1, gp_W2, gp_b2, alpha):
    return pl.pallas_call(
        _combine_body,
        out_shape=(
            jax.ShapeDtypeStruct((B, 1), jnp.float32),
            jax.ShapeDtypeStruct((B, 1), jnp.float32),
        ),
    )(out_struct_raw, out_feat, gp_W1, gp_b1.reshape(1, HID),
      gp_W2, gp_b2.reshape(1, 1), alpha.reshape(1, 2))


def _mlp(x, W1, b1, W2, b2):
    return jax.nn.relu(x @ W1 + b1) @ W2 + b2


def _gcn_conv(x, row, col, W, b):
    n = x.shape[0]
    x = x @ W
    sl = jnp.arange(n, dtype=row.dtype)
    r = jnp.concatenate([row, sl])
    c = jnp.concatenate([col, sl])
    deg = jax.ops.segment_sum(jnp.ones(r.shape[0], x.dtype), c, num_segments=n)
    dinv = jax.lax.rsqrt(jnp.maximum(deg, 1e-12))
    norm = dinv[r] * dinv[c]
    msg = x[c] * norm[:, None]
    return jax.ops.segment_sum(msg, r, num_segments=n) + b


def kernel(x, edge_index, edge, A_values, W0, b0, W1, b1, W2, b2, fe_W1,
           fe_b1, fe_W2, fe_b2, fn_W1, fn_b1, fn_W2, fn_b2, gp_W1, gp_b1,
           gp_W2, gp_b2, alpha):
    row, col = edge_index[0], edge_index[1]
    h = jax.nn.relu(_gcn_conv(x, row, col, W0, b0))
    h = jax.nn.relu(_gcn_conv(h, row, col, W1, b1))
    h = _gcn_conv(h, row, col, W2, b2)
    e0, e1 = edge[0], edge[1]
    out_feat = jnp.sum(h[e0] * h[e1], axis=-1, keepdims=True)

    ew = _mlp(A_values[:, None], fe_W1, fe_b1, fe_W2, fe_b2)
    nsf = jax.ops.segment_sum(ew, col, num_segments=N)
    fn = _mlp(nsf, fn_W1, fn_b1, fn_W2, fn_b2)[:, 0]

    A_dense = jnp.zeros((N, N), jnp.float32).at[row, col].add(A_values)
    mat_src = A_dense[e0] * fn[None, :]
    mat_dst = A_dense[e1] * fn[None, :]
    out_struct_raw = jnp.sum(mat_src * mat_dst, axis=1, keepdims=True)

    out, out_struct = _combine(out_struct_raw, out_feat, gp_W1, gp_b1,
                               gp_W2, gp_b2, alpha)
    return out, out_struct, out_feat
